# baseline (speedup 1.0000x reference)
import functools

import jax
import jax.numpy as jnp
from jax import lax
from jax.experimental import pallas as pl
from jax.experimental.pallas import tpu as pltpu
from jax.experimental.pallas import tpu_sc as plsc

_D = 64
_B = 16384
_VROWS = 500224

_NC = 2
_NS = 16
_NW = _NC * _NS
_BPW = _B // _NW


def _sc_gather_pairs(idx_a, idx_b, table2):
    mesh = plsc.VectorSubcoreMesh(core_axis_name="c", subcore_axis_name="s")

    @functools.partial(
        pl.kernel,
        mesh=mesh,
        out_type=[
            jax.ShapeDtypeStruct((_B, 2 * _D), jnp.float32),
            jax.ShapeDtypeStruct((_B, 2 * _D), jnp.float32),
        ],
        scratch_types=[
            pltpu.VMEM((_BPW,), jnp.int32),
            pltpu.VMEM((_BPW, 2 * _D), jnp.float32),
            pltpu.SemaphoreType.DMA,
        ],
    )
    def gather_kernel(idx_a_hbm, idx_b_hbm, table_hbm, out_a_hbm, out_b_hbm,
                      idx_v, rows_v, sem_r):
        wid = lax.axis_index("s") * _NC + lax.axis_index("c")
        base = wid * _BPW

        def one_side(idx_hbm, out_hbm):
            pltpu.sync_copy(idx_hbm.at[pl.ds(base, _BPW)], idx_v)

            def to_row(g, carry):
                v = idx_v[pl.ds(g * 16, 16)]
                row = jnp.left_shift(jnp.right_shift(v, 10), 9) | (v & 511)
                idx_v[pl.ds(g * 16, 16)] = row
                return carry

            lax.fori_loop(0, _BPW // 16, to_row, 0)
            pltpu.async_copy(table_hbm.at[idx_v], rows_v, sem_r).wait()
            pltpu.sync_copy(rows_v, out_hbm.at[pl.ds(base, _BPW)])

        one_side(idx_a_hbm, out_a_hbm)
        one_side(idx_b_hbm, out_b_hbm)

    return gather_kernel(idx_a, idx_b, table2)


def _mlp_body(pa_ref, pb_ref, ia_ref, ib_ref, w1a_ref, w1b_ref, b1_ref,
              w2_ref, b2_ref, w3_ref, b3_ref, o_ref):
    pa = (jax.lax.shift_right_logical(ia_ref[...], 9) & 1).astype(jnp.float32)
    pb = (jax.lax.shift_right_logical(ib_ref[...], 9) & 1).astype(jnp.float32)
    a = pa * pa_ref[:, _D:] + (1.0 - pa) * pa_ref[:, :_D]
    b = pb * pb_ref[:, _D:] + (1.0 - pb) * pb_ref[:, :_D]
    h = jnp.dot(a, w1a_ref[...], preferred_element_type=jnp.float32)
    h = h + jnp.dot(b, w1b_ref[...], preferred_element_type=jnp.float32)
    h = jnp.maximum(h + b1_ref[...], 0.0)
    h = jnp.dot(h, w2_ref[...], preferred_element_type=jnp.float32)
    h = jnp.maximum(h + b2_ref[...], 0.0)
    o = jnp.dot(h, w3_ref[...], preferred_element_type=jnp.float32) + b3_ref[...]
    o_ref[...] = jax.nn.sigmoid(o)


def _tc_mlp(emb2_a, emb2_b, idx_a2, idx_b2, w1a, w1b, b1, w2, b2, w3, b3, blk):
    grid = _B // blk
    full = lambda i: (0, 0)
    return pl.pallas_call(
        _mlp_body,
        grid=(grid,),
        in_specs=[
            pl.BlockSpec((blk, 2 * _D), lambda i: (i, 0)),
            pl.BlockSpec((blk, 2 * _D), lambda i: (i, 0)),
            pl.BlockSpec((blk, 1), lambda i: (i, 0)),
            pl.BlockSpec((blk, 1), lambda i: (i, 0)),
            pl.BlockSpec((_D, 128), full),
            pl.BlockSpec((_D, 128), full),
            pl.BlockSpec((1, 128), full),
            pl.BlockSpec((128, _D), full),
            pl.BlockSpec((1, _D), full),
            pl.BlockSpec((_D, 1), full),
            pl.BlockSpec((1, 1), full),
        ],
        out_specs=pl.BlockSpec((blk, 1), lambda i: (i, 0)),
        out_shape=jax.ShapeDtypeStruct((_B, 1), jnp.float32),
    )(emb2_a, emb2_b, idx_a2, idx_b2, w1a, w1b, b1, w2, b2, w3, b3)


_PBLK = 512


def _pack_body(ttl_ref, ttr_ref, o_ref):
    eye = jnp.eye(_D, dtype=jnp.float32)
    btl = jax.lax.dot_general(
        ttl_ref[...], eye,
        dimension_numbers=(((0,), (0,)), ((), ())),
        preferred_element_type=jnp.float32)
    btr = jax.lax.dot_general(
        ttr_ref[...], eye,
        dimension_numbers=(((0,), (0,)), ((), ())),
        preferred_element_type=jnp.float32)
    o_ref[:, :_D] = btl
    o_ref[:, _D:] = btr


def _tc_pack(table_t):
    grid = (1000000 + 2 * _PBLK - 1) // (2 * _PBLK)
    return pl.pallas_call(
        _pack_body,
        grid=(grid,),
        in_specs=[
            pl.BlockSpec((_D, _PBLK), lambda s: (0, 2 * s)),
            pl.BlockSpec((_D, _PBLK), lambda s: (0, 2 * s + 1)),
        ],
        out_specs=pl.BlockSpec((_PBLK, 2 * _D), lambda s: (s, 0)),
        out_shape=jax.ShapeDtypeStruct((_VROWS, 2 * _D), jnp.float32),
    )(table_t, table_t)


def kernel(drug_a_idx, drug_b_idx, table, W1, b1, W2, b2, W3, b3):
    idx_a = drug_a_idx.astype(jnp.int32)
    idx_b = drug_b_idx.astype(jnp.int32)
    table2 = _tc_pack(table.T)
    emb2_a, emb2_b = _sc_gather_pairs(idx_a, idx_b, table2)
    w1a = W1[:, :_D].T
    w1b = W1[:, _D:].T
    return _tc_mlp(emb2_a, emb2_b,
                   idx_a.reshape(_B, 1), idx_b.reshape(_B, 1),
                   w1a, w1b, b1.reshape(1, 128), W2.T, b2.reshape(1, _D),
                   W3.T, b3.reshape(1, 1), blk=2048)


# speedup vs baseline: 1.0453x; 1.0453x over previous
import functools

import jax
import jax.numpy as jnp
from jax import lax
from jax.experimental import pallas as pl
from jax.experimental.pallas import tpu as pltpu
from jax.experimental.pallas import tpu_sc as plsc

_D = 64
_B = 16384
_VROWS = 500224

_NC = 2
_NS = 16
_NW = _NC * _NS
_BPW = _B // _NW


def _sc_gather_pairs(idx_a, idx_b, table2):
    mesh = plsc.VectorSubcoreMesh(core_axis_name="c", subcore_axis_name="s")

    @functools.partial(
        pl.kernel,
        mesh=mesh,
        out_type=[
            jax.ShapeDtypeStruct((_B, 2 * _D), jnp.float32),
            jax.ShapeDtypeStruct((_B, 2 * _D), jnp.float32),
        ],
        scratch_types=[
            pltpu.VMEM((_BPW,), jnp.int32),
            pltpu.VMEM((_BPW, 2 * _D), jnp.float32),
            pltpu.SemaphoreType.DMA,
        ],
    )
    def gather_kernel(idx_a_hbm, idx_b_hbm, table_hbm, out_a_hbm, out_b_hbm,
                      idx_v, rows_v, sem_r):
        wid = lax.axis_index("s") * _NC + lax.axis_index("c")
        base = wid * _BPW

        def one_side(idx_hbm, out_hbm):
            pltpu.sync_copy(idx_hbm.at[pl.ds(base, _BPW)], idx_v)

            def to_row(g, carry):
                v = idx_v[pl.ds(g * 16, 16)]
                row = jnp.left_shift(jnp.right_shift(v, 10), 9) | (v & 511)
                idx_v[pl.ds(g * 16, 16)] = row
                return carry

            lax.fori_loop(0, _BPW // 16, to_row, 0)
            pltpu.async_copy(table_hbm.at[idx_v], rows_v, sem_r).wait()
            pltpu.sync_copy(rows_v, out_hbm.at[pl.ds(base, _BPW)])

        one_side(idx_a_hbm, out_a_hbm)
        one_side(idx_b_hbm, out_b_hbm)

    return gather_kernel(idx_a, idx_b, table2)


def _mlp_body(pa_ref, pb_ref, ia_ref, ib_ref, w1a_ref, w1b_ref, b1_ref,
              w2_ref, b2_ref, w3_ref, b3_ref, o_ref):
    pa = (jax.lax.shift_right_logical(ia_ref[...], 9) & 1).astype(jnp.float32)
    pb = (jax.lax.shift_right_logical(ib_ref[...], 9) & 1).astype(jnp.float32)
    a = pa * pa_ref[:, _D:] + (1.0 - pa) * pa_ref[:, :_D]
    b = pb * pb_ref[:, _D:] + (1.0 - pb) * pb_ref[:, :_D]
    h = jnp.dot(a, w1a_ref[...], preferred_element_type=jnp.float32)
    h = h + jnp.dot(b, w1b_ref[...], preferred_element_type=jnp.float32)
    h = jnp.maximum(h + b1_ref[...], 0.0)
    h = jnp.dot(h, w2_ref[...], preferred_element_type=jnp.float32)
    h = jnp.maximum(h + b2_ref[...], 0.0)
    o = jnp.dot(h, w3_ref[...], preferred_element_type=jnp.float32) + b3_ref[...]
    o_ref[...] = jax.nn.sigmoid(o)


def _tc_mlp(emb2_a, emb2_b, idx_a2, idx_b2, w1a, w1b, b1, w2, b2, w3, b3, blk):
    grid = _B // blk
    full = lambda i: (0, 0)
    return pl.pallas_call(
        _mlp_body,
        grid=(grid,),
        in_specs=[
            pl.BlockSpec((blk, 2 * _D), lambda i: (i, 0)),
            pl.BlockSpec((blk, 2 * _D), lambda i: (i, 0)),
            pl.BlockSpec((blk, 1), lambda i: (i, 0)),
            pl.BlockSpec((blk, 1), lambda i: (i, 0)),
            pl.BlockSpec((_D, 128), full),
            pl.BlockSpec((_D, 128), full),
            pl.BlockSpec((1, 128), full),
            pl.BlockSpec((128, _D), full),
            pl.BlockSpec((1, _D), full),
            pl.BlockSpec((_D, 1), full),
            pl.BlockSpec((1, 1), full),
        ],
        out_specs=pl.BlockSpec((blk, 1), lambda i: (i, 0)),
        out_shape=jax.ShapeDtypeStruct((_B, 1), jnp.float32),
    )(emb2_a, emb2_b, idx_a2, idx_b2, w1a, w1b, b1, w2, b2, w3, b3)


_PBLK = 512


def _pack_body(ttl_ref, ttr_ref, o_ref):
    o_ref[:, :_D] = ttl_ref[...].T
    o_ref[:, _D:] = ttr_ref[...].T


def _tc_pack(table_t):
    grid = (1000000 + 2 * _PBLK - 1) // (2 * _PBLK)
    return pl.pallas_call(
        _pack_body,
        grid=(grid,),
        in_specs=[
            pl.BlockSpec((_D, _PBLK), lambda s: (0, 2 * s)),
            pl.BlockSpec((_D, _PBLK), lambda s: (0, 2 * s + 1)),
        ],
        out_specs=pl.BlockSpec((_PBLK, 2 * _D), lambda s: (s, 0)),
        out_shape=jax.ShapeDtypeStruct((_VROWS, 2 * _D), jnp.float32),
    )(table_t, table_t)


def kernel(drug_a_idx, drug_b_idx, table, W1, b1, W2, b2, W3, b3):
    idx_a = drug_a_idx.astype(jnp.int32)
    idx_b = drug_b_idx.astype(jnp.int32)
    table2 = _tc_pack(table.T)
    emb2_a, emb2_b = _sc_gather_pairs(idx_a, idx_b, table2)
    w1a = W1[:, :_D].T
    w1b = W1[:, _D:].T
    return _tc_mlp(emb2_a, emb2_b,
                   idx_a.reshape(_B, 1), idx_b.reshape(_B, 1),
                   w1a, w1b, b1.reshape(1, 128), W2.T, b2.reshape(1, _D),
                   W3.T, b3.reshape(1, 1), blk=2048)


# speedup vs baseline: 2.0195x; 1.9320x over previous
import functools

import jax
import jax.numpy as jnp
from jax import lax
from jax.experimental import pallas as pl
from jax.experimental.pallas import tpu as pltpu
from jax.experimental.pallas import tpu_sc as plsc

_D = 64
_B = 16384

_NC = 2
_NS = 16
_NW = _NC * _NS
_BPW = _B // _NW


def _sc_gather(idx_a, idx_b, table):
    mesh = plsc.VectorSubcoreMesh(core_axis_name="c", subcore_axis_name="s")

    @functools.partial(
        pl.kernel,
        mesh=mesh,
        out_type=[
            jax.ShapeDtypeStruct((_B, _D), jnp.float32),
            jax.ShapeDtypeStruct((_B, _D), jnp.float32),
        ],
        scratch_types=[
            pltpu.VMEM((_BPW,), jnp.int32),
            pltpu.VMEM((_BPW, _D), jnp.float32),
            pltpu.SemaphoreType.DMA,
        ],
    )
    def gather_kernel(idx_a_hbm, idx_b_hbm, table_hbm, out_a_hbm, out_b_hbm,
                      idx_v, rows_v, sem_r):
        wid = lax.axis_index("s") * _NC + lax.axis_index("c")
        base = wid * _BPW

        def one_side(idx_hbm, out_hbm):
            pltpu.sync_copy(idx_hbm.at[pl.ds(base, _BPW)], idx_v)

            def issue(g, carry):
                vec = idx_v[pl.ds(g * 16, 16)]
                for j in range(16):
                    r = vec[j]
                    pltpu.make_async_copy(
                        table_hbm.at[r], rows_v.at[g * 16 + j], sem_r).start()
                return carry

            lax.fori_loop(0, _BPW // 16, issue, 0)

            def drain(i, carry):
                pltpu.make_async_copy(
                    table_hbm.at[0], rows_v.at[i], sem_r).wait()
                return carry

            lax.fori_loop(0, _BPW, drain, 0)
            pltpu.sync_copy(rows_v, out_hbm.at[pl.ds(base, _BPW)])

        one_side(idx_a_hbm, out_a_hbm)
        one_side(idx_b_hbm, out_b_hbm)

    return gather_kernel(idx_a, idx_b, table)


def _mlp_body(a_ref, b_ref, w1a_ref, w1b_ref, b1_ref, w2_ref, b2_ref,
              w3_ref, b3_ref, o_ref):
    h = jnp.dot(a_ref[...], w1a_ref[...], preferred_element_type=jnp.float32)
    h = h + jnp.dot(b_ref[...], w1b_ref[...], preferred_element_type=jnp.float32)
    h = jnp.maximum(h + b1_ref[...], 0.0)
    h = jnp.dot(h, w2_ref[...], preferred_element_type=jnp.float32)
    h = jnp.maximum(h + b2_ref[...], 0.0)
    o = jnp.dot(h, w3_ref[...], preferred_element_type=jnp.float32) + b3_ref[...]
    o_ref[...] = jax.nn.sigmoid(o)


def _tc_mlp(emb_a, emb_b, w1a, w1b, b1, w2, b2, w3, b3, blk):
    grid = _B // blk
    full = lambda i: (0, 0)
    return pl.pallas_call(
        _mlp_body,
        grid=(grid,),
        in_specs=[
            pl.BlockSpec((blk, _D), lambda i: (i, 0)),
            pl.BlockSpec((blk, _D), lambda i: (i, 0)),
            pl.BlockSpec((_D, 128), full),
            pl.BlockSpec((_D, 128), full),
            pl.BlockSpec((1, 128), full),
            pl.BlockSpec((128, _D), full),
            pl.BlockSpec((1, _D), full),
            pl.BlockSpec((_D, 1), full),
            pl.BlockSpec((1, 1), full),
        ],
        out_specs=pl.BlockSpec((blk, 1), lambda i: (i, 0)),
        out_shape=jax.ShapeDtypeStruct((_B, 1), jnp.float32),
    )(emb_a, emb_b, w1a, w1b, b1, w2, b2, w3, b3)


def kernel(drug_a_idx, drug_b_idx, table, W1, b1, W2, b2, W3, b3):
    idx_a = drug_a_idx.astype(jnp.int32)
    idx_b = drug_b_idx.astype(jnp.int32)
    emb_a, emb_b = _sc_gather(idx_a, idx_b, table)
    w1a = W1[:, :_D].T
    w1b = W1[:, _D:].T
    return _tc_mlp(emb_a, emb_b, w1a, w1b,
                   b1.reshape(1, 128), W2.T, b2.reshape(1, _D),
                   W3.T, b3.reshape(1, 1), blk=2048)
